# baseline (speedup 1.0000x reference)
import functools

import jax
import jax.numpy as jnp
from jax import lax
from jax.experimental import pallas as pl
from jax.experimental.pallas import tpu as pltpu
from jax.experimental.pallas import tpu_sc as plsc

EMBED = 64
NSEG = 3
L = 16
NC = 2
NS = 16
NW = NC * NS
SEQ = 200
BBLK = 16384 // NW
NOCT = SEQ // 8


def _tec_body(ids_hbm, tab_hbm, out_hbm,
              iv0, iv1, ov0, ov1, tabt_v, tab_v,
              isem0, isem1, osem0, osem1):
    wid = lax.axis_index("s") * NC + lax.axis_index("c")
    b0 = wid * BBLK

    pltpu.sync_copy(tab_hbm, tab_v)
    rowsel = jnp.minimum(lax.iota(jnp.int32, L), NSEG - 1) * EMBED

    def prep_e(e, carry):
        tabt_v[pl.ds(e * L, L)] = plsc.load_gather(tab_v, [rowsel + e])
        return carry

    lax.fori_loop(0, EMBED, prep_e, 0, unroll=False)

    def compute_s(ids_v, si, ts, ov, osem):
        od = ts * 8 + si

        @pl.when(od >= 2)
        def _wait_out():
            pltpu.make_async_copy(
                ov, out_hbm.at[0, :, pl.ds(b0, BBLK)], osem).wait()

        def bg_body(bg, carry):
            ids16 = ids_v[si, pl.ds(bg * L, L)]
            for e in range(EMBED):
                col = tabt_v[pl.ds(e * L, L)]
                ov[e, pl.ds(bg * L, L)] = jnp.take_along_axis(
                    col, ids16, axis=0)
            return carry

        lax.fori_loop(0, BBLK // L, bg_body, 0, unroll=False)
        pltpu.async_copy(ov, out_hbm.at[od, :, pl.ds(b0, BBLK)], osem)

    def octet(ts, ids_v):
        def sp_body(sp, carry):
            compute_s(ids_v, 2 * sp, ts, ov0, osem0)
            compute_s(ids_v, 2 * sp + 1, ts, ov1, osem1)
            return carry

        lax.fori_loop(0, 4, sp_body, 0, unroll=False)

    def ids_dma(ts, iv, isem):
        return pltpu.async_copy(
            ids_hbm.at[pl.ds(ts * 8, 8), pl.ds(b0, BBLK)], iv, isem)

    def ids_wait(iv, isem):
        pltpu.make_async_copy(
            ids_hbm.at[pl.ds(0, 8), pl.ds(b0, BBLK)], iv, isem).wait()

    ids_dma(0, iv0, isem0)
    ids_dma(1, iv1, isem1)

    def pair_body(p, carry):
        ids_wait(iv0, isem0)
        octet(2 * p, iv0)
        ids_dma(2 * p + 2, iv0, isem0)

        ids_wait(iv1, isem1)
        octet(2 * p + 1, iv1)

        @pl.when(p < (NOCT - 1) // 2 - 1)
        def _pf1():
            ids_dma(2 * p + 3, iv1, isem1)
        return carry

    lax.fori_loop(0, (NOCT - 1) // 2, pair_body, 0, unroll=False)

    ids_wait(iv0, isem0)
    octet(NOCT - 1, iv0)

    pltpu.make_async_copy(ov0, out_hbm.at[0, :, pl.ds(b0, BBLK)],
                          osem0).wait()
    pltpu.make_async_copy(ov1, out_hbm.at[0, :, pl.ds(b0, BBLK)],
                          osem1).wait()


@jax.jit
def _sc_lookup(ids_t, tab_flat):
    mesh = plsc.VectorSubcoreMesh(core_axis_name="c", subcore_axis_name="s")
    kfn = pl.kernel(
        _tec_body,
        out_type=jax.ShapeDtypeStruct((SEQ, EMBED, 16384), jnp.float32),
        mesh=mesh,
        scratch_types=[
            pltpu.VMEM((8, BBLK), jnp.int32),
            pltpu.VMEM((8, BBLK), jnp.int32),
            pltpu.VMEM((EMBED, BBLK), jnp.float32),
            pltpu.VMEM((EMBED, BBLK), jnp.float32),
            pltpu.VMEM((EMBED * L,), jnp.float32),
            pltpu.VMEM((NSEG * EMBED,), jnp.float32),
            pltpu.SemaphoreType.DMA,
            pltpu.SemaphoreType.DMA,
            pltpu.SemaphoreType.DMA,
            pltpu.SemaphoreType.DMA,
        ],
        compiler_params=pltpu.CompilerParams(needs_layout_passes=False),
    )
    return kfn(ids_t, tab_flat)


def kernel(segment_ids, table):
    b, s = segment_ids.shape
    ids_t = segment_ids.astype(jnp.int32).T
    tab_flat = table.reshape(NSEG * EMBED)
    out_t = _sc_lookup(ids_t, tab_flat)
    return out_t.transpose(2, 0, 1)


# speedup vs baseline: 6.2025x; 6.2025x over previous
import functools

import jax
import jax.numpy as jnp
from jax import lax
from jax.experimental import pallas as pl
from jax.experimental.pallas import tpu as pltpu
from jax.experimental.pallas import tpu_sc as plsc

EMBED = 64
NSEG = 3
L = 16
NC = 2
NS = 16
NW = NC * NS
SEQ = 200
BBLK = 16384 // NW
NOCT = SEQ // 8


def _tec_body(ids_hbm, tab_hbm, out_hbm,
              iv0, iv1, ov0, ov1, tabt_v, tab_v,
              isem0, isem1, osem0, osem1):
    wid = lax.axis_index("s") * NC + lax.axis_index("c")
    b0 = wid * BBLK

    pltpu.sync_copy(tab_hbm, tab_v)
    rowsel = jnp.minimum(lax.iota(jnp.int32, L), NSEG - 1) * EMBED

    def prep_e(e, carry):
        tabt_v[pl.ds(e * L, L)] = plsc.load_gather(tab_v, [rowsel + e])
        return carry

    lax.fori_loop(0, EMBED, prep_e, 0, unroll=False)

    def compute_s(ids_v, si, ts, ov, osem):
        od = ts * 8 + si

        @pl.when(od >= 2)
        def _wait_out():
            pltpu.make_async_copy(
                ov, out_hbm.at[0, :, pl.ds(b0, BBLK)], osem).wait()

        for half in range(2):
            e0 = half * (EMBED // 2)
            cols = [tabt_v[pl.ds((e0 + k) * L, L)] for k in range(EMBED // 2)]

            @plsc.parallel_loop(0, BBLK // L, unroll=2)
            def bg_body(bg):
                ids16 = ids_v[si, pl.ds(bg * L, L)]
                for k in range(EMBED // 2):
                    ov[e0 + k, pl.ds(bg * L, L)] = jnp.take_along_axis(
                        cols[k], ids16, axis=0, mode="promise_in_bounds")
        pltpu.async_copy(ov, out_hbm.at[od, :, pl.ds(b0, BBLK)], osem)

    def octet(ts, ids_v):
        def sp_body(sp, carry):
            compute_s(ids_v, 2 * sp, ts, ov0, osem0)
            compute_s(ids_v, 2 * sp + 1, ts, ov1, osem1)
            return carry

        lax.fori_loop(0, 4, sp_body, 0, unroll=False)

    def ids_dma(ts, iv, isem):
        return pltpu.async_copy(
            ids_hbm.at[pl.ds(ts * 8, 8), pl.ds(b0, BBLK)], iv, isem)

    def ids_wait(iv, isem):
        pltpu.make_async_copy(
            ids_hbm.at[pl.ds(0, 8), pl.ds(b0, BBLK)], iv, isem).wait()

    ids_dma(0, iv0, isem0)
    ids_dma(1, iv1, isem1)

    def pair_body(p, carry):
        ids_wait(iv0, isem0)
        octet(2 * p, iv0)
        ids_dma(2 * p + 2, iv0, isem0)

        ids_wait(iv1, isem1)
        octet(2 * p + 1, iv1)

        @pl.when(p < (NOCT - 1) // 2 - 1)
        def _pf1():
            ids_dma(2 * p + 3, iv1, isem1)
        return carry

    lax.fori_loop(0, (NOCT - 1) // 2, pair_body, 0, unroll=False)

    ids_wait(iv0, isem0)
    octet(NOCT - 1, iv0)

    pltpu.make_async_copy(ov0, out_hbm.at[0, :, pl.ds(b0, BBLK)],
                          osem0).wait()
    pltpu.make_async_copy(ov1, out_hbm.at[0, :, pl.ds(b0, BBLK)],
                          osem1).wait()


@jax.jit
def _sc_lookup(ids_t, tab_flat):
    mesh = plsc.VectorSubcoreMesh(core_axis_name="c", subcore_axis_name="s")
    kfn = pl.kernel(
        _tec_body,
        out_type=jax.ShapeDtypeStruct((SEQ, EMBED, 16384), jnp.float32),
        mesh=mesh,
        scratch_types=[
            pltpu.VMEM((8, BBLK), jnp.int32),
            pltpu.VMEM((8, BBLK), jnp.int32),
            pltpu.VMEM((EMBED, BBLK), jnp.float32),
            pltpu.VMEM((EMBED, BBLK), jnp.float32),
            pltpu.VMEM((EMBED * L,), jnp.float32),
            pltpu.VMEM((NSEG * EMBED,), jnp.float32),
            pltpu.SemaphoreType.DMA,
            pltpu.SemaphoreType.DMA,
            pltpu.SemaphoreType.DMA,
            pltpu.SemaphoreType.DMA,
        ],
        compiler_params=pltpu.CompilerParams(needs_layout_passes=False),
    )
    return kfn(ids_t, tab_flat)


def kernel(segment_ids, table):
    b, s = segment_ids.shape
    ids_t = segment_ids.astype(jnp.int32).T
    tab_flat = table.reshape(NSEG * EMBED)
    out_t = _sc_lookup(ids_t, tab_flat)
    return out_t.transpose(2, 0, 1)
